# baseline (speedup 1.0000x reference)
import jax
import jax.numpy as jnp
from jax import lax
from jax.experimental import pallas as pl
from jax.experimental.pallas import tpu as pltpu
from jax.experimental.pallas import tpu_sc as plsc

OUT_W = 272
OUT_WP = 384
IN_W = 456
SCALE = 1.6764705882352942
INV_SUPPORT = 0.5964912280701754
NTAPS = 4

B, C, H = 64, 3, 456
R = B * C * H
NWORKERS = 32
ROWS_PER_W = R // NWORKERS
CHUNK = 24
NCHUNKS = ROWS_PER_W // CHUNK
PLANES_PER_W = (B * C) // NWORKERS
CHUNKS_PER_PLANE = H // CHUNK
NOVEC = OUT_W // 16


def _tap_tables(og):
    f32, i32 = jnp.float32, jnp.int32
    o = (lax.iota(i32, 16) + og * 16).astype(f32)
    center = (o + 0.5) * SCALE
    lowi = jnp.maximum((center - SCALE + 0.5).astype(i32), 0)
    highi = jnp.minimum((center + SCALE + 0.5).astype(i32), IN_W)
    width = jnp.minimum(highi - lowi, 5)
    lowf = lowi.astype(f32)
    ws, idxs = [], []
    for j in range(NTAPS):
        dist = (lowf + float(j) - center + 0.5) * INV_SUPPORT
        wj = 1.0 - jnp.minimum(jnp.abs(dist), 1.0)
        wj = jnp.where(width > j, wj, 0.0)
        ws.append(wj)
        idxs.append(jnp.minimum(lowi + j, IN_W - 1))
    wsum = (ws[0] + ws[1]) + (ws[2] + ws[3])
    ws = [w / wsum for w in ws]
    return idxs, ws


def _resize_body(x_hbm, out_hbm, idx_t, w_t, in_v0, in_v1, out_v0, out_v1,
                 sin0, sin1, sout0, sout1):
    wid = lax.axis_index("s") * 2 + lax.axis_index("c")
    base_plane = wid * PLANES_PER_W
    in_bufs = (in_v0, in_v1)
    out_bufs = (out_v0, out_v1)
    sins = (sin0, sin1)
    souts = (sout0, sout1)

    for og in range(NOVEC):
        idxs, ws = _tap_tables(og)
        for j in range(NTAPS):
            idx_t[j, pl.ds(og * 16, 16)] = idxs[j]
            w_t[j, pl.ds(og * 16, 16)] = ws[j]

    def plane_row(ci):
        plane = base_plane + ci // CHUNKS_PER_PLANE
        r0 = (ci % CHUNKS_PER_PLANE) * CHUNK
        return plane // C, plane % C, r0

    def in_copy(ci, b):
        bi, ch, r0 = plane_row(ci)
        return pltpu.make_async_copy(
            x_hbm.at[bi, ch, pl.ds(r0, CHUNK), :], in_bufs[b], sins[b])

    def out_copy(ci, b):
        bi, ch, r0 = plane_row(ci)
        return pltpu.make_async_copy(
            out_bufs[b], out_hbm.at[bi, ch, pl.ds(r0, CHUNK), :], souts[b])

    in_copy(0, 0).start()

    def outer(cc, carry):
        for b in range(2):
            ci = cc * 2 + b

            @pl.when(ci + 1 < NCHUNKS)
            def _():
                in_copy(ci + 1, 1 - b).start()

            in_copy(ci, b).wait()

            @pl.when(cc >= 1)
            def _():
                out_copy(ci - 2, b).wait()

            def og_body(og, c2, b=b):
                colv = [idx_t[j, pl.ds(og * 16, 16)] for j in range(NTAPS)]
                wv = [w_t[j, pl.ds(og * 16, 16)] for j in range(NTAPS)]
                for k in range(0, CHUNK, 8):
                    gs = []
                    for d in range(8):
                        rv = jnp.full((16,), k + d, dtype=jnp.int32)
                        gs.append([plsc.load_gather(in_bufs[b], [rv, cv])
                                   for cv in colv])
                    for d in range(8):
                        g = gs[d]
                        acc = (g[0] * wv[0] + g[1] * wv[1]) + (g[2] * wv[2] + g[3] * wv[3])
                        out_bufs[b][k + d, pl.ds(og * 16, 16)] = acc
                return c2

            lax.fori_loop(0, NOVEC, og_body, 0)

            out_copy(ci, b).start()
        return carry

    lax.fori_loop(0, NCHUNKS // 2, outer, 0)
    out_copy(NCHUNKS - 2, 0).wait()
    out_copy(NCHUNKS - 1, 1).wait()


@jax.jit
def _resize(x4d):
    mesh = plsc.VectorSubcoreMesh(core_axis_name="c", subcore_axis_name="s")
    return pl.kernel(
        _resize_body,
        out_type=jax.ShapeDtypeStruct((B, C, H, OUT_WP), jnp.float32),
        mesh=mesh,
        compiler_params=pltpu.CompilerParams(needs_layout_passes=False),
        scratch_types=[
            pltpu.VMEM((NTAPS, OUT_W), jnp.int32),
            pltpu.VMEM((NTAPS, OUT_W), jnp.float32),
            pltpu.VMEM((CHUNK, IN_W), jnp.float32),
            pltpu.VMEM((CHUNK, IN_W), jnp.float32),
            pltpu.VMEM((CHUNK, OUT_WP), jnp.float32),
            pltpu.VMEM((CHUNK, OUT_WP), jnp.float32),
            pltpu.SemaphoreType.DMA,
            pltpu.SemaphoreType.DMA,
            pltpu.SemaphoreType.DMA,
            pltpu.SemaphoreType.DMA,
        ],
    )(x4d)


def kernel(arg0_1):
    outp = _resize(arg0_1)
    return (outp[..., :OUT_W],)


# speedup vs baseline: 1.0121x; 1.0121x over previous
import jax
import jax.numpy as jnp
from jax import lax
from jax.experimental import pallas as pl
from jax.experimental.pallas import tpu as pltpu
from jax.experimental.pallas import tpu_sc as plsc

OUT_W = 272
OUT_WP = 384
IN_W = 456
SCALE = 1.6764705882352942
INV_SUPPORT = 0.5964912280701754
NTAPS = 4

B, C, H = 64, 3, 456
R = B * C * H
NWORKERS = 32
ROWS_PER_W = R // NWORKERS
CHUNK = 24
NCHUNKS = ROWS_PER_W // CHUNK
PLANES_PER_W = (B * C) // NWORKERS
CHUNKS_PER_PLANE = H // CHUNK
NOVEC = OUT_W // 16


def _tap_tables(og):
    f32, i32 = jnp.float32, jnp.int32
    o = (lax.iota(i32, 16) + og * 16).astype(f32)
    center = (o + 0.5) * SCALE
    lowi = jnp.maximum((center - SCALE + 0.5).astype(i32), 0)
    highi = jnp.minimum((center + SCALE + 0.5).astype(i32), IN_W)
    width = jnp.minimum(highi - lowi, 5)
    lowf = lowi.astype(f32)
    ws, idxs = [], []
    for j in range(NTAPS):
        dist = (lowf + float(j) - center + 0.5) * INV_SUPPORT
        wj = 1.0 - jnp.minimum(jnp.abs(dist), 1.0)
        wj = jnp.where(width > j, wj, 0.0)
        ws.append(wj)
        idxs.append(jnp.minimum(lowi + j, IN_W - 1))
    wsum = (ws[0] + ws[1]) + (ws[2] + ws[3])
    ws = [w / wsum for w in ws]
    return idxs, ws


def _resize_body(x_hbm, out_hbm, idx_t, w_t, in_v0, in_v1, out_v0, out_v1,
                 sin0, sin1, sout0, sout1):
    wid = lax.axis_index("s") * 2 + lax.axis_index("c")
    base_plane = wid * PLANES_PER_W
    in_bufs = (in_v0, in_v1)
    out_bufs = (out_v0, out_v1)
    sins = (sin0, sin1)
    souts = (sout0, sout1)

    for og in range(NOVEC):
        idxs, ws = _tap_tables(og)
        for j in range(NTAPS):
            idx_t[j, pl.ds(og * 16, 16)] = idxs[j]
            w_t[j, pl.ds(og * 16, 16)] = ws[j]

    def plane_row(ci):
        plane = base_plane + ci // CHUNKS_PER_PLANE
        r0 = (ci % CHUNKS_PER_PLANE) * CHUNK
        return plane // C, plane % C, r0

    def in_copy(ci, b):
        bi, ch, r0 = plane_row(ci)
        return pltpu.make_async_copy(
            x_hbm.at[bi, ch, pl.ds(r0, CHUNK), :], in_bufs[b], sins[b])

    def out_copy(ci, b):
        bi, ch, r0 = plane_row(ci)
        return pltpu.make_async_copy(
            out_bufs[b], out_hbm.at[bi, ch, pl.ds(r0, CHUNK), :], souts[b])

    in_copy(0, 0).start()

    def outer(cc, carry):
        for b in range(2):
            ci = cc * 2 + b

            @pl.when(ci + 1 < NCHUNKS)
            def _():
                in_copy(ci + 1, 1 - b).start()

            in_copy(ci, b).wait()

            @pl.when(cc >= 1)
            def _():
                out_copy(ci - 2, b).wait()

            def og_body(og, c2, b=b):
                colv = [idx_t[j, pl.ds(og * 16, 16)] for j in range(NTAPS)]
                wv = [w_t[j, pl.ds(og * 16, 16)] for j in range(NTAPS)]
                for k in range(0, CHUNK, 12):
                    gs = []
                    for d in range(12):
                        rv = jnp.full((16,), k + d, dtype=jnp.int32)
                        gs.append([plsc.load_gather(in_bufs[b], [rv, cv])
                                   for cv in colv])
                    for d in range(12):
                        g = gs[d]
                        acc = (g[0] * wv[0] + g[1] * wv[1]) + (g[2] * wv[2] + g[3] * wv[3])
                        out_bufs[b][k + d, pl.ds(og * 16, 16)] = acc
                return c2

            lax.fori_loop(0, NOVEC, og_body, 0)

            out_copy(ci, b).start()
        return carry

    lax.fori_loop(0, NCHUNKS // 2, outer, 0)
    out_copy(NCHUNKS - 2, 0).wait()
    out_copy(NCHUNKS - 1, 1).wait()


@jax.jit
def _resize(x4d):
    mesh = plsc.VectorSubcoreMesh(core_axis_name="c", subcore_axis_name="s")
    return pl.kernel(
        _resize_body,
        out_type=jax.ShapeDtypeStruct((B, C, H, OUT_WP), jnp.float32),
        mesh=mesh,
        compiler_params=pltpu.CompilerParams(needs_layout_passes=False),
        scratch_types=[
            pltpu.VMEM((NTAPS, OUT_W), jnp.int32),
            pltpu.VMEM((NTAPS, OUT_W), jnp.float32),
            pltpu.VMEM((CHUNK, IN_W), jnp.float32),
            pltpu.VMEM((CHUNK, IN_W), jnp.float32),
            pltpu.VMEM((CHUNK, OUT_WP), jnp.float32),
            pltpu.VMEM((CHUNK, OUT_WP), jnp.float32),
            pltpu.SemaphoreType.DMA,
            pltpu.SemaphoreType.DMA,
            pltpu.SemaphoreType.DMA,
            pltpu.SemaphoreType.DMA,
        ],
    )(x4d)


def kernel(arg0_1):
    outp = _resize(arg0_1)
    return (outp[..., :OUT_W],)
